# Initial kernel scaffold; baseline (speedup 1.0000x reference)
import jax
import jax.numpy as jnp
from jax.experimental import pallas as pl


def kernel(x, edge_index, W0, b0, W1, b1, W2, b2, W3, b3, Wl, bl):
    raise NotImplementedError("write your pallas kernel here")


# speedup vs baseline: 9.7918x; 9.7918x over previous
import functools

import jax
import jax.numpy as jnp
from jax import lax
from jax.experimental import pallas as pl
from jax.experimental.pallas import tpu as pltpu
from jax.experimental.pallas import tpu_sc as plsc

N = 10000
E = 320000
D = 128
NCLS = 40

NC = 2
NS = 16
NW = NC * NS
CHUNK = 128
NBLK = -(-E // (NW * CHUNK))
EPW = NBLK * CHUNK
EPAD = EPW * NW
RPT = 640
ACC_ROWS = NS * RPT
NROWS = ACC_ROWS
NPAD = NROWS
TRASH = 10008
RB = 1024
NRB = NROWS // RB

_mesh = plsc.VectorSubcoreMesh(core_axis_name="c", subcore_axis_name="s")
_sc_params = pltpu.CompilerParams(needs_layout_passes=False)


@functools.partial(
    pl.kernel,
    out_type=jax.ShapeDtypeStruct((NW, NPAD), jnp.float32),
    mesh=_mesh,
    scratch_types=[
        pltpu.VMEM((NBLK, CHUNK), jnp.int32),
        pltpu.VMEM((NPAD,), jnp.float32),
    ],
    compiler_params=_sc_params,
)
def _sc_deg(dst_hbm, out_hbm, dst_v, deg_v):
    c = lax.axis_index("c")
    s = lax.axis_index("s")
    w = c * NS + s
    zero16 = jnp.zeros((16,), jnp.float32)
    ones16 = jnp.ones((16,), jnp.float32)

    def _zero(i, _):
        deg_v[pl.ds(i * 16, 16)] = zero16
        return 0

    lax.fori_loop(0, NPAD // 16, _zero, 0)
    pltpu.sync_copy(dst_hbm.at[w], dst_v)

    def _blk(j, _):
        for kk in range(CHUNK // 16):
            idx = dst_v[j, pl.ds(kk * 16, 16)]
            plsc.addupdate_scatter(deg_v, [idx], ones16)
        return 0

    lax.fori_loop(0, NBLK, _blk, 0)
    pltpu.sync_copy(deg_v, out_hbm.at[w])


@functools.partial(
    pl.kernel,
    out_type=jax.ShapeDtypeStruct((NC, ACC_ROWS, D), jnp.float32),
    mesh=_mesh,
    scratch_types=[
        pltpu.VMEM((NBLK, CHUNK), jnp.int32),
        pltpu.VMEM((NBLK, CHUNK), jnp.int32),
        pltpu.VMEM((CHUNK, D), jnp.float32),
        pltpu.VMEM_SHARED((ACC_ROWS, D), jnp.float32),
    ],
    compiler_params=_sc_params,
)
def _sc_scatter(hpre_hbm, src_hbm, dst_hbm, out_hbm, src_v, dst_v, rows_v, acc_sh):
    c = lax.axis_index("c")
    s = lax.axis_index("s")
    w = c * NS + s
    zero16 = jnp.zeros((16,), jnp.float32)

    def _zrow(i, _):
        for jj in range(D // 16):
            rows_v[i, pl.ds(jj * 16, 16)] = zero16
        return 0

    lax.fori_loop(0, CHUNK, _zrow, 0)
    for k in range(RPT // CHUNK):
        pltpu.sync_copy(rows_v, acc_sh.at[pl.ds(s * RPT + k * CHUNK, CHUNK)])
    plsc.subcore_barrier()

    pltpu.sync_copy(src_hbm.at[w], src_v)
    pltpu.sync_copy(dst_hbm.at[w], dst_v)

    def _blk(j, _):
        pltpu.sync_copy(hpre_hbm.at[src_v.at[j]], rows_v)
        pltpu.sync_copy(rows_v, acc_sh.at[dst_v.at[j]], add=True)
        return 0

    lax.fori_loop(0, NBLK, _blk, 0)
    plsc.subcore_barrier()

    for k in range(RPT // CHUNK):
        r0 = s * RPT + k * CHUNK
        pltpu.sync_copy(acc_sh.at[pl.ds(r0, CHUNK)], rows_v)
        pltpu.sync_copy(rows_v, out_hbm.at[c, pl.ds(r0, CHUNK)])


def _tc_prep_body(degp_ref, x_ref, w0_ref, dinv_ref, hpre_ref):
    deg = jnp.sum(degp_ref[...], axis=0) + 1.0
    dv = lax.rsqrt(deg)[:, None]
    dinv_ref[...] = jnp.broadcast_to(dv, (RB, D))
    h = jnp.dot(x_ref[...], w0_ref[...], preferred_element_type=jnp.float32)
    hpre_ref[...] = h * dv


_tc_prep = pl.pallas_call(
    _tc_prep_body,
    grid=(NRB,),
    in_specs=[
        pl.BlockSpec((NW, RB), lambda i: (0, i)),
        pl.BlockSpec((RB, D), lambda i: (i, 0)),
        pl.BlockSpec((D, D), lambda i: (0, 0)),
    ],
    out_specs=[
        pl.BlockSpec((RB, D), lambda i: (i, 0)),
        pl.BlockSpec((RB, D), lambda i: (i, 0)),
    ],
    out_shape=[
        jax.ShapeDtypeStruct((NROWS, D), jnp.float32),
        jax.ShapeDtypeStruct((NROWS, D), jnp.float32),
    ],
)


def _tc_combine_body(accs_ref, hpre_ref, dinv_ref, m_ref, b_ref, wn_ref,
                     mout_ref, hnext_ref):
    a = accs_ref[0] + accs_ref[1]
    dv = dinv_ref[...]
    tot = (a + hpre_ref[...]) * dv + b_ref[...][0:1, :]
    h = jnp.maximum(tot, 0.0)
    mout_ref[...] = jnp.maximum(m_ref[...], h)
    hnext_ref[...] = jnp.dot(h, wn_ref[...], preferred_element_type=jnp.float32) * dv


_tc_combine = pl.pallas_call(
    _tc_combine_body,
    grid=(NRB,),
    in_specs=[
        pl.BlockSpec((NC, RB, D), lambda i: (0, i, 0)),
        pl.BlockSpec((RB, D), lambda i: (i, 0)),
        pl.BlockSpec((RB, D), lambda i: (i, 0)),
        pl.BlockSpec((RB, D), lambda i: (i, 0)),
        pl.BlockSpec((8, D), lambda i: (0, 0)),
        pl.BlockSpec((D, D), lambda i: (0, 0)),
    ],
    out_specs=[
        pl.BlockSpec((RB, D), lambda i: (i, 0)),
        pl.BlockSpec((RB, D), lambda i: (i, 0)),
    ],
    out_shape=[
        jax.ShapeDtypeStruct((NROWS, D), jnp.float32),
        jax.ShapeDtypeStruct((NROWS, D), jnp.float32),
    ],
)


def _tc_final_body(accs_ref, hpre_ref, dinv_ref, m_ref, b_ref, wl_ref, bl_ref,
                   out_ref):
    a = accs_ref[0] + accs_ref[1]
    tot = (a + hpre_ref[...]) * dinv_ref[...] + b_ref[...][0:1, :]
    h = jnp.maximum(tot, 0.0)
    m = jnp.maximum(m_ref[...], h)
    out_ref[...] = (jnp.dot(m, wl_ref[...], preferred_element_type=jnp.float32)
                    + bl_ref[...][0:1, :])


_tc_final = pl.pallas_call(
    _tc_final_body,
    grid=(NRB,),
    in_specs=[
        pl.BlockSpec((NC, RB, D), lambda i: (0, i, 0)),
        pl.BlockSpec((RB, D), lambda i: (i, 0)),
        pl.BlockSpec((RB, D), lambda i: (i, 0)),
        pl.BlockSpec((RB, D), lambda i: (i, 0)),
        pl.BlockSpec((8, D), lambda i: (0, 0)),
        pl.BlockSpec((D, D), lambda i: (0, 0)),
        pl.BlockSpec((8, D), lambda i: (0, 0)),
    ],
    out_specs=pl.BlockSpec((RB, D), lambda i: (i, 0)),
    out_shape=jax.ShapeDtypeStruct((NROWS, D), jnp.float32),
)


def kernel(x, edge_index, W0, b0, W1, b1, W2, b2, W3, b3, Wl, bl):
    src = edge_index[0].astype(jnp.int32)
    dst = edge_index[1].astype(jnp.int32)
    pad = EPAD - E
    src3 = jnp.concatenate([src, jnp.zeros((pad,), jnp.int32)]).reshape(
        NW, NBLK, CHUNK)
    dst3 = jnp.concatenate([dst, jnp.full((pad,), TRASH, jnp.int32)]).reshape(
        NW, NBLK, CHUNK)

    b8 = [jnp.broadcast_to(b.reshape(1, D), (8, D)) for b in (b0, b1, b2, b3)]
    wl_p = jnp.pad(Wl, ((0, 0), (0, D - NCLS)))
    bl_p = jnp.broadcast_to(jnp.pad(bl, (0, D - NCLS)).reshape(1, D), (8, D))
    x_p = jnp.pad(x, ((0, NROWS - N), (0, 0)))

    degp = _sc_deg(dst3)
    dinv, hpre = _tc_prep(degp, x_p, W0)

    m = jnp.zeros((NROWS, D), jnp.float32)
    w_next = [W1, W2, W3]
    for i in range(3):
        accs = _sc_scatter(hpre, src3, dst3)
        m, hpre = _tc_combine(accs, hpre, dinv, m, b8[i], w_next[i])
    accs = _sc_scatter(hpre, src3, dst3)
    out = _tc_final(accs, hpre, dinv, m, b8[3], wl_p, bl_p)
    return out[:N, :NCLS]
